# initial kernel scaffold (unmeasured)
import jax
import jax.numpy as jnp
from jax import lax
from jax.experimental import pallas as pl
from jax.experimental.pallas import tpu as pltpu


def kernel(
    x,
):
    def body(*refs):
        pass

    out_shape = jax.ShapeDtypeStruct(..., jnp.float32)
    return pl.pallas_call(body, out_shape=out_shape)(...)


# baseline (device time: 65351 ns/iter reference)
import jax
import jax.numpy as jnp
from jax import lax
from jax.experimental import pallas as pl
from jax.experimental.pallas import tpu as pltpu

N_DEV = 32


def kernel(x):
    m_per, n = x.shape

    def body(x_ref, out_ref, comm_ref, send_sems, recv_sems):
        my = lax.axis_index("i")
        left = lax.rem(my - 1 + N_DEV, N_DEV)
        right = lax.rem(my + 1, N_DEV)

        barrier_sem = pltpu.get_barrier_semaphore()
        for nbr in (left, right):
            pl.semaphore_signal(
                barrier_sem, inc=1,
                device_id=(nbr,), device_id_type=pl.DeviceIdType.MESH,
            )
        pl.semaphore_wait(barrier_sem, 2)

        xv = x_ref[:, :]
        mv = jnp.max(xv, axis=0)
        row_ids = lax.broadcasted_iota(jnp.int32, (m_per, n), 0)
        local_idx = jnp.min(
            jnp.where(xv == mv[None, :], row_ids, m_per), axis=0
        )
        gidx = (local_idx + my * m_per).astype(jnp.float32)
        comm_ref[0, 0, :] = mv
        comm_ref[0, 1, :] = gidx

        for h in range(N_DEV - 1):
            rdma = pltpu.make_async_remote_copy(
                src_ref=comm_ref.at[h],
                dst_ref=comm_ref.at[h + 1],
                send_sem=send_sems.at[h],
                recv_sem=recv_sems.at[h],
                device_id=(right,),
                device_id_type=pl.DeviceIdType.MESH,
            )
            rdma.start()
            rdma.wait()

        parts = comm_ref[:, :, :]
        vals = parts[:, 0, :]
        idxs = parts[:, 1, :]
        best_v = jnp.max(vals, axis=0)
        best_i = jnp.min(
            jnp.where(vals == best_v[None, :], idxs, jnp.float32(3.4e38)),
            axis=0,
        )
        out_ref[0, :] = best_v
        out_ref[1, :] = best_i

    return pl.pallas_call(
        body,
        out_shape=jax.ShapeDtypeStruct((2, n), jnp.float32),
        in_specs=[pl.BlockSpec(memory_space=pltpu.VMEM)],
        out_specs=pl.BlockSpec(memory_space=pltpu.VMEM),
        scratch_shapes=[
            pltpu.VMEM((N_DEV, 2, n), jnp.float32),
            pltpu.SemaphoreType.DMA((N_DEV - 1,)),
            pltpu.SemaphoreType.DMA((N_DEV - 1,)),
        ],
        compiler_params=pltpu.CompilerParams(collective_id=0),
    )(x)


# device time: 18460 ns/iter; 3.5401x vs baseline; 3.5401x over previous
import jax
import jax.numpy as jnp
from jax import lax
from jax.experimental import pallas as pl
from jax.experimental.pallas import tpu as pltpu

N_DEV = 32
LOG2_N = 5


def kernel(x):
    m_per, n = x.shape

    def body(x_ref, out_ref, send_buf, recv_buf, send_sems, recv_sems):
        my = lax.axis_index("i")

        barrier_sem = pltpu.get_barrier_semaphore()
        for k in range(LOG2_N):
            pl.semaphore_signal(
                barrier_sem, inc=1,
                device_id=(my ^ (1 << k),),
                device_id_type=pl.DeviceIdType.MESH,
            )
        pl.semaphore_wait(barrier_sem, LOG2_N)

        xv = x_ref[:, :]
        acc_v = jnp.max(xv, axis=0)
        row_ids = lax.broadcasted_iota(jnp.int32, (m_per, n), 0)
        local_idx = jnp.min(
            jnp.where(xv == acc_v[None, :], row_ids, m_per), axis=0
        )
        acc_i = (local_idx + my * m_per).astype(jnp.float32)

        for k in range(LOG2_N):
            send_buf[k, 0, :] = acc_v
            send_buf[k, 1, :] = acc_i
            rdma = pltpu.make_async_remote_copy(
                src_ref=send_buf.at[k],
                dst_ref=recv_buf.at[k],
                send_sem=send_sems.at[k],
                recv_sem=recv_sems.at[k],
                device_id=(my ^ (1 << k),),
                device_id_type=pl.DeviceIdType.MESH,
            )
            rdma.start()
            rdma.wait()
            ov = recv_buf[k, 0, :]
            oi = recv_buf[k, 1, :]
            take = (ov > acc_v) | ((ov == acc_v) & (oi < acc_i))
            acc_v = jnp.where(take, ov, acc_v)
            acc_i = jnp.where(take, oi, acc_i)

        out_ref[0, :] = acc_v
        out_ref[1, :] = acc_i

    return pl.pallas_call(
        body,
        out_shape=jax.ShapeDtypeStruct((2, n), jnp.float32),
        in_specs=[pl.BlockSpec(memory_space=pltpu.VMEM)],
        out_specs=pl.BlockSpec(memory_space=pltpu.VMEM),
        scratch_shapes=[
            pltpu.VMEM((LOG2_N, 2, n), jnp.float32),
            pltpu.VMEM((LOG2_N, 2, n), jnp.float32),
            pltpu.SemaphoreType.DMA((LOG2_N,)),
            pltpu.SemaphoreType.DMA((LOG2_N,)),
        ],
        compiler_params=pltpu.CompilerParams(collective_id=0),
    )(x)


# device time: 12976 ns/iter; 5.0363x vs baseline; 1.4226x over previous
import jax
import jax.numpy as jnp
from jax import lax
from jax.experimental import pallas as pl
from jax.experimental.pallas import tpu as pltpu

N_DEV = 32


def kernel(x):
    m_per, n = x.shape

    def body(x_ref, out_ref, comm_ref, send_buf, send_sems, recv_sems):
        my = lax.axis_index("i")

        barrier_sem = pltpu.get_barrier_semaphore()
        for off in range(1, N_DEV):
            pl.semaphore_signal(
                barrier_sem, inc=1,
                device_id=(lax.rem(my + off, N_DEV),),
                device_id_type=pl.DeviceIdType.MESH,
            )

        xv = x_ref[:, :]
        acc_v = jnp.max(xv, axis=0)
        row_ids = lax.broadcasted_iota(jnp.int32, (m_per, n), 0)
        local_idx = jnp.min(
            jnp.where(xv == acc_v[None, :], row_ids, m_per), axis=0
        )
        acc_i = (local_idx + my * m_per).astype(jnp.float32)
        send_buf[0, :] = acc_v
        send_buf[1, :] = acc_i
        comm_ref[0, 0, :] = acc_v
        comm_ref[0, 1, :] = acc_i

        pl.semaphore_wait(barrier_sem, N_DEV - 1)

        rdmas = []
        for off in range(1, N_DEV):
            rdma = pltpu.make_async_remote_copy(
                src_ref=send_buf,
                dst_ref=comm_ref.at[off],
                send_sem=send_sems.at[off],
                recv_sem=recv_sems.at[off],
                device_id=(lax.rem(my + off, N_DEV),),
                device_id_type=pl.DeviceIdType.MESH,
            )
            rdma.start()
            rdmas.append(rdma)

        for rdma in rdmas:
            rdma.wait_recv()

        parts = comm_ref[:, :, :]
        vals = parts[:, 0, :]
        idxs = parts[:, 1, :]
        best_v = jnp.max(vals, axis=0)
        best_i = jnp.min(
            jnp.where(vals == best_v[None, :], idxs, jnp.float32(3.4e38)),
            axis=0,
        )
        out_ref[0, :] = best_v
        out_ref[1, :] = best_i

        for rdma in rdmas:
            rdma.wait_send()

    return pl.pallas_call(
        body,
        out_shape=jax.ShapeDtypeStruct((2, n), jnp.float32),
        in_specs=[pl.BlockSpec(memory_space=pltpu.VMEM)],
        out_specs=pl.BlockSpec(memory_space=pltpu.VMEM),
        scratch_shapes=[
            pltpu.VMEM((N_DEV, 2, n), jnp.float32),
            pltpu.VMEM((2, n), jnp.float32),
            pltpu.SemaphoreType.DMA((N_DEV,)),
            pltpu.SemaphoreType.DMA((N_DEV,)),
        ],
        compiler_params=pltpu.CompilerParams(collective_id=0),
    )(x)
